# Initial kernel scaffold; baseline (speedup 1.0000x reference)
import jax
import jax.numpy as jnp
from jax.experimental import pallas as pl


def kernel(table_pad, dlogp_pad, query, pos_items, seed_key):
    raise NotImplementedError("write your pallas kernel here")


# speedup vs baseline: 39.0354x; 39.0354x over previous
import functools

import numpy as np
import jax
import jax.numpy as jnp
from jax.experimental import pallas as pl
from jax.experimental.pallas import tpu as pltpu

_LANES = 128
_S = 512
_W = 640
_NEG_SENTINEL = 1 << 30


def _ceil_to(x, m):
    return -(-x // m) * m


def _sample_body(coarse_ref, comb_ref, seeds_ref, items_ref, prob_ref, *, k, r):
    seeds = seeds_ref[...]
    coarse = coarse_ref[...].reshape(1, 1, k)
    lt = (coarse < seeds[:, :, None]).astype(jnp.float32)
    b = jnp.minimum(jnp.sum(lt, axis=-1), float(k - 1))
    bi = b.astype(jnp.int32)
    iota = jax.lax.broadcasted_iota(jnp.int32, (r, _LANES, k), 2)
    onehot = (iota == bi[:, :, None]).astype(jnp.float32)
    onehot2 = onehot.reshape(r * _LANES, k)
    g = jnp.dot(onehot2, comb_ref[...], preferred_element_type=jnp.float32)
    g3 = g.reshape(r, _LANES, 2 * _W)
    cmp = g3[..., :_W]
    val = g3[..., _W:]
    mask = (cmp < seeds[:, :, None]).astype(jnp.float32)
    cnt = jnp.sum(mask, axis=-1) - 1.0
    psum = jnp.sum(mask * val, axis=-1)
    items_ref[...] = (b * _S + cnt).astype(jnp.int32)
    prob_ref[...] = psum


def _pos_body(comb_ref, items_ref, prob_ref, *, k, r):
    items = items_ref[...]
    b = jnp.minimum(items // _S, k - 1)
    local = items - b * _S
    iota = jax.lax.broadcasted_iota(jnp.int32, (r, _LANES, k), 2)
    onehot = (iota == b[:, :, None]).astype(jnp.float32)
    onehot2 = onehot.reshape(r * _LANES, k)
    g = jnp.dot(onehot2, comb_ref[...], preferred_element_type=jnp.float32)
    g3 = g.reshape(r, _LANES, _W)
    ji = jax.lax.broadcasted_iota(jnp.int32, (1, 1, _W), 2)
    jcmp = jnp.where(ji == _S, -1, jnp.where(ji > _S, _NEG_SENTINEL, ji))
    mask = (jcmp < local[:, :, None]).astype(jnp.float32)
    prob_ref[...] = jnp.sum(mask * g3, axis=-1)


def _build_tables(table_pad, dlogp_pad):
    npad = table_pad.shape[-1]
    npb = _ceil_to(npad, _S)
    t = table_pad.reshape(-1)
    d = dlogp_pad.reshape(-1)
    if npb != npad:
        t = jnp.pad(t, (0, npb - npad), constant_values=2.0)
        d = jnp.pad(d, (0, npb - npad))
    k = npb // _S
    t2 = t.reshape(k, _S)
    d2 = d.reshape(k, _S)
    coarse = t2[:, -1].reshape(1, k)
    bsum = jnp.cumsum(jnp.sum(d2, axis=1))
    base = jnp.concatenate([jnp.zeros((1,), jnp.float32), bsum[:-1]])
    neg1 = jnp.full((k, 1), -1.0, jnp.float32)
    two = jnp.full((k, _W - _S - 1), 2.0, jnp.float32)
    zpad = jnp.zeros((k, _W - _S - 1), jnp.float32)
    comb = jnp.concatenate(
        [t2, neg1, two, d2, base[:, None], zpad], axis=1)
    comb_pos = jnp.concatenate([d2, base[:, None], zpad], axis=1)
    return k, coarse, comb, comb_pos


def _tile_rows(flat, r):
    m = flat.shape[0]
    rows = max(1, _ceil_to(m, _LANES) // _LANES)
    rows_pad = _ceil_to(rows, r)
    total = rows_pad * _LANES
    if total != m:
        flat = jnp.pad(flat, (0, total - m))
    return flat.reshape(rows_pad, _LANES), rows_pad


def kernel(table_pad, dlogp_pad, query, pos_items, seed_key):
    k, coarse, comb, comb_pos = _build_tables(table_pad, dlogp_pad)

    q_prefix = query.shape[:-1]
    num_queries = int(np.prod(q_prefix))
    num_neg = 32
    key = jax.random.wrap_key_data(seed_key)
    seeds = jax.random.uniform(key, (num_queries, num_neg), dtype=jnp.float32)

    r = 16
    m = num_queries * num_neg
    seeds2d, rows_pad = _tile_rows(seeds.reshape(-1), r)
    items2d, prob2d = pl.pallas_call(
        functools.partial(_sample_body, k=k, r=r),
        out_shape=(jax.ShapeDtypeStruct((rows_pad, _LANES), jnp.int32),
                   jax.ShapeDtypeStruct((rows_pad, _LANES), jnp.float32)),
        grid=(rows_pad // r,),
        in_specs=[pl.BlockSpec((1, k), lambda i: (0, 0)),
                  pl.BlockSpec((k, 2 * _W), lambda i: (0, 0)),
                  pl.BlockSpec((r, _LANES), lambda i: (i, 0))],
        out_specs=[pl.BlockSpec((r, _LANES), lambda i: (i, 0)),
                   pl.BlockSpec((r, _LANES), lambda i: (i, 0))],
        compiler_params=pltpu.CompilerParams(
            dimension_semantics=("parallel",),
            vmem_limit_bytes=64 * 1024 * 1024),
    )(coarse, comb, seeds2d)
    neg_items = items2d.reshape(-1)[:m].reshape(*q_prefix, num_neg)
    neg_prob = prob2d.reshape(-1)[:m].reshape(*q_prefix, num_neg)

    rp = 16
    mp = int(np.prod(pos_items.shape))
    pos2d, prows_pad = _tile_rows(pos_items.reshape(-1).astype(jnp.int32), rp)
    pprob2d = pl.pallas_call(
        functools.partial(_pos_body, k=k, r=rp),
        out_shape=jax.ShapeDtypeStruct((prows_pad, _LANES), jnp.float32),
        grid=(prows_pad // rp,),
        in_specs=[pl.BlockSpec((k, _W), lambda i: (0, 0)),
                  pl.BlockSpec((rp, _LANES), lambda i: (i, 0))],
        out_specs=pl.BlockSpec((rp, _LANES), lambda i: (i, 0)),
        compiler_params=pltpu.CompilerParams(
            dimension_semantics=("parallel",),
            vmem_limit_bytes=64 * 1024 * 1024),
    )(comb_pos, pos2d)
    pos_prob = pprob2d.reshape(-1)[:mp].reshape(pos_items.shape)

    return pos_prob, neg_items, neg_prob
